# baseline (speedup 1.0000x reference)
import functools

import jax
import jax.numpy as jnp
from jax import lax
from jax.experimental import pallas as pl
from jax.experimental.pallas import tpu as pltpu
from jax.experimental.pallas import tpu_sc as plsc

_GCHUNK = 512
_IDXBLK = 2 * _GCHUNK


def _make_detile_tc(V, D):
    C = 512
    H = 500224
    grid = H // C

    dn = (((0,), (0,)), ((), ()))

    def body(lo_ref, hi_ref, eye_ref, out_ref):
        lo = lo_ref[...]
        hi = hi_ref[...]
        eye = eye_ref[...]
        yt_lo = jax.lax.dot_general(
            lo, eye, dn, preferred_element_type=jnp.float32
        )
        yt_hi = jax.lax.dot_general(
            hi, eye, dn, preferred_element_type=jnp.float32
        )
        out_ref[...] = jnp.concatenate([yt_lo, yt_hi], axis=1)

    call = pl.pallas_call(
        body,
        out_shape=jax.ShapeDtypeStruct((H, 2 * D), jnp.float32),
        grid=(grid,),
        in_specs=[
            pl.BlockSpec((D, C), lambda j: (0, j)),
            pl.BlockSpec((D, C), lambda j: (0, j + H // C)),
            pl.BlockSpec((D, D), lambda j: (0, 0)),
        ],
        out_specs=pl.BlockSpec((C, 2 * D), lambda j: (j, 0)),
    )
    return lambda wt: call(wt, wt, jnp.eye(D, dtype=jnp.float32))


def _make_detile(V, D):
    info = plsc.get_sparse_core_info()
    NC, NS = info.num_cores, info.num_subcores
    NW = NC * NS
    n_full = V // 128
    tail = V - n_full * 128
    per_w2 = (n_full // NW + 2) // 2
    DS = D + 1
    BLK = 128 * DS

    mesh = plsc.VectorSubcoreMesh(core_axis_name="c", subcore_axis_name="s")

    @functools.partial(
        pl.kernel,
        out_type=jax.ShapeDtypeStruct((V * DS,), jnp.float32),
        mesh=mesh,
        scratch_types=[
            pltpu.VMEM((D, 128), jnp.float32),
            pltpu.VMEM((D, 128), jnp.float32),
            pltpu.VMEM((BLK,), jnp.float32),
            pltpu.VMEM((BLK,), jnp.float32),
            pltpu.VMEM((64, D), jnp.float32),
            pltpu.SemaphoreType.DMA,
            pltpu.SemaphoreType.DMA,
        ],
        compiler_params=pltpu.CompilerParams(needs_layout_passes=False),
    )
    def detile_kernel(wt_hbm, wtail_hbm, w1_hbm, in0, in1, ov0, ov1, tb, ssem, wsem):
        wid = lax.axis_index("s") * NC + lax.axis_index("c")
        iota = lax.iota(jnp.int32, 16)
        skiota = iota * DS
        inbufs = (in0, in1)
        ovbufs = (ov0, ov1)

        def stage(jj, buf):
            v0 = pl.multiple_of(jj * 128, 128)
            return pltpu.async_copy(
                wt_hbm.at[pl.ds(0, D), pl.ds(v0, 128)], buf, ssem
            )

        @pl.when(wid < n_full)
        def _():
            stage(wid, in0)

        def blk2(i2, carry):
            for p in range(2):
                i = 2 * i2 + p
                j = wid + i * NW

                @pl.when(j < n_full)
                def _():
                    pltpu.make_async_copy(
                        wt_hbm.at[pl.ds(0, D), pl.ds(0, 128)],
                        inbufs[p],
                        ssem,
                    ).wait()

                @pl.when(j + NW < n_full)
                def _():
                    stage(j + NW, inbufs[1 - p])

                @pl.when(jnp.logical_and(i >= 2, j - 2 * NW < n_full))
                def _():
                    pltpu.make_async_copy(
                        ovbufs[p], w1_hbm.at[pl.ds(0, BLK)], wsem
                    ).wait()

                @pl.when(j < n_full)
                def _():
                    inb, ov = inbufs[p], ovbufs[p]

                    def g_body(g, carry2):
                        base = g * (16 * DS)
                        for d in range(D):
                            vreg = inb[d, pl.ds(g * 16, 16)]
                            plsc.store_scatter(ov, [skiota + (base + d)], vreg)
                        return carry2

                    lax.fori_loop(0, 8, g_body, 0)
                    o0 = pl.multiple_of(j * BLK, BLK)
                    pltpu.async_copy(ov, w1_hbm.at[pl.ds(o0, BLK)], wsem)

            return carry

        lax.fori_loop(0, per_w2, blk2, 0)
        for k in (2 * per_w2 - 2, 2 * per_w2 - 1):
            @pl.when(wid + k * NW < n_full)
            def _():
                pltpu.make_async_copy(
                    ovbufs[k % 2], w1_hbm.at[pl.ds(0, BLK)], wsem
                ).wait()

        if tail:
            @pl.when(wid == 1)
            def _():
                pltpu.sync_copy(wtail_hbm, tb)

                def t_body(r, carry2):
                    for g in range(D // 16):
                        ov0[pl.ds(r * DS + g * 16, 16)] = tb[r, pl.ds(g * 16, 16)]
                    return carry2

                lax.fori_loop(0, tail, t_body, 0)
                pltpu.sync_copy(
                    ov0.at[pl.ds(0, tail * DS)],
                    w1_hbm.at[pl.ds(n_full * BLK, tail * DS)],
                )

    return detile_kernel


def _make_gather(V, D, N):
    info = plsc.get_sparse_core_info()
    NC, NS = info.num_cores, info.num_subcores
    NW = NC * NS
    assert N % (NW * _IDXBLK) == 0
    n_per_w = N // NW
    n_it = n_per_w // _IDXBLK

    mesh = plsc.VectorSubcoreMesh(core_axis_name="c", subcore_axis_name="s")

    @functools.partial(
        pl.kernel,
        out_type=jax.ShapeDtypeStruct((N, 2 * D), jnp.float32),
        mesh=mesh,
        scratch_types=[
            pltpu.VMEM((_IDXBLK,), jnp.int32),
            pltpu.VMEM((_GCHUNK, D + 1), jnp.float32),
            pltpu.VMEM((_GCHUNK, D + 1), jnp.float32),
            pltpu.SemaphoreType.DMA,
            pltpu.SemaphoreType.DMA,
        ],
        compiler_params=pltpu.CompilerParams(use_tc_tiling_on_sc=False),
    )
    def gather_kernel(w_hbm, idx_hbm, out_hbm, idx_buf, rows0, rows1, gsem, wsem):
        wid = lax.axis_index("s") * NC + lax.axis_index("c")
        wbase = wid * n_per_w
        bufs = (rows0, rows1)

        def body(i, carry):
            base = pl.multiple_of(wbase + i * _IDXBLK, _IDXBLK)
            pltpu.sync_copy(idx_hbm.at[pl.ds(base, _IDXBLK)], idx_buf)
            for s in range(2):
                buf = bufs[s]
                @pl.when(i > 0)
                def _():
                    pltpu.make_async_copy(
                        buf.at[pl.ds(0, _GCHUNK), pl.ds(0, D)],
                        out_hbm.at[pl.ds(0, _GCHUNK), pl.ds(0, D)],
                        wsem,
                    ).wait()
                pltpu.async_copy(
                    w_hbm.at[idx_buf.at[pl.ds(s * _GCHUNK, _GCHUNK)]],
                    buf,
                    gsem,
                ).wait()
                pltpu.async_copy(
                    buf.at[pl.ds(0, _GCHUNK), pl.ds(0, D)],
                    out_hbm.at[pl.ds(base + s * _GCHUNK, _GCHUNK), pl.ds(0, D)],
                    wsem,
                )
            return carry

        lax.fori_loop(0, n_it, body, 0)
        for s in range(2):
            pltpu.make_async_copy(
                bufs[s].at[pl.ds(0, _GCHUNK), pl.ds(0, D)],
                out_hbm.at[pl.ds(0, _GCHUNK), pl.ds(0, D)],
                wsem,
            ).wait()

    return gather_kernel


def kernel(q, W):
    B, L = q.shape
    V, D = W.shape
    N = B * L
    idx = q.reshape(N).astype(jnp.int32)
    tail = V - (V // 128) * 128
    w1 = _make_detile(V, D)(W.T, W[V - tail:, :])
    out = _make_gather(V, D, N)(w1.reshape(V, D + 1), idx)
    return out[:, :D].reshape(B, L, D)


# speedup vs baseline: 2.6263x; 2.6263x over previous
import functools

import jax
import jax.numpy as jnp
from jax import lax
from jax.experimental import pallas as pl
from jax.experimental.pallas import tpu as pltpu
from jax.experimental.pallas import tpu_sc as plsc

_GCHUNK = 512
_IDXBLK = 2 * _GCHUNK


def _make_gather(V, D, N):
    info = plsc.get_sparse_core_info()
    NC, NS = info.num_cores, info.num_subcores
    NW = NC * NS
    assert N % (NW * _IDXBLK) == 0
    n_per_w = N // NW
    n_it = n_per_w // _IDXBLK

    mesh = plsc.VectorSubcoreMesh(core_axis_name="c", subcore_axis_name="s")

    @functools.partial(
        pl.kernel,
        out_type=jax.ShapeDtypeStruct((N, 2 * D), jnp.float32),
        mesh=mesh,
        scratch_types=[
            pltpu.VMEM((_IDXBLK,), jnp.int32),
            pltpu.VMEM((_GCHUNK, D), jnp.float32),
            pltpu.VMEM((_GCHUNK, D), jnp.float32),
            pltpu.SemaphoreType.DMA,
            pltpu.SemaphoreType.DMA,
        ],
        compiler_params=pltpu.CompilerParams(use_tc_tiling_on_sc=False),
    )
    def gather_kernel(w_hbm, idx_hbm, out_hbm, idx_buf, rows0, rows1, gsem, wsem):
        wid = lax.axis_index("s") * NC + lax.axis_index("c")
        wbase = wid * n_per_w
        bufs = (rows0, rows1)

        def body(i, carry):
            base = pl.multiple_of(wbase + i * _IDXBLK, _IDXBLK)
            pltpu.sync_copy(idx_hbm.at[pl.ds(base, _IDXBLK)], idx_buf)
            for s in range(2):
                buf = bufs[s]
                @pl.when(i > 0)
                def _():
                    pltpu.make_async_copy(
                        buf, out_hbm.at[pl.ds(0, _GCHUNK), pl.ds(0, D)], wsem
                    ).wait()
                pltpu.async_copy(
                    w_hbm.at[idx_buf.at[pl.ds(s * _GCHUNK, _GCHUNK)]],
                    buf,
                    gsem,
                ).wait()
                pltpu.async_copy(
                    buf,
                    out_hbm.at[pl.ds(base + s * _GCHUNK, _GCHUNK), pl.ds(0, D)],
                    wsem,
                )
            return carry

        lax.fori_loop(0, n_it, body, 0)
        for s in range(2):
            pltpu.make_async_copy(
                bufs[s], out_hbm.at[pl.ds(0, _GCHUNK), pl.ds(0, D)], wsem
            ).wait()

    return gather_kernel


def kernel(q, W):
    B, L = q.shape
    V, D = W.shape
    N = B * L
    idx = q.reshape(N).astype(jnp.int32)
    out = _make_gather(V, D, N)(W, idx)
    return out[:, :D].reshape(B, L, D)
